# baseline (speedup 1.0000x reference)
import functools

import jax
import jax.numpy as jnp
from jax import lax
from jax.experimental import pallas as pl
from jax.experimental.pallas import tpu as pltpu
from jax.experimental.pallas import tpu_sc as plsc

_DIM = 64
_LANES = 16
_NW = 32
_BLK = 128
_V = 1000000
_VMAIN = 999936
_NBLK = 7812
_IDX_CHUNK = 128


def _relayout_kernel():
    mesh = plsc.VectorSubcoreMesh(core_axis_name="c", subcore_axis_name="s")
    n_base = _NBLK // _NW
    n_last = _NBLK - 31 * n_base

    @functools.partial(
        pl.kernel,
        out_type=jax.ShapeDtypeStruct((_V * _DIM,), jnp.float32),
        mesh=mesh,
        scratch_types=[
            pltpu.VMEM((_DIM, _BLK), jnp.float32),
            pltpu.VMEM((_DIM, _BLK), jnp.float32),
            pltpu.VMEM((_BLK * _DIM,), jnp.float32),
            pltpu.VMEM((_BLK * _DIM,), jnp.float32),
            pltpu.VMEM((_DIM * _DIM,), jnp.float32),
            pltpu.SemaphoreType.DMA,
            pltpu.SemaphoreType.DMA,
            pltpu.SemaphoreType.DMA,
            pltpu.SemaphoreType.DMA,
        ],
        compiler_params=pltpu.CompilerParams(
            needs_layout_passes=False, use_tc_tiling_on_sc=True),
    )
    def kern(tab_hbm, tail_hbm, out_hbm, tba, tbb, roa, rob, tailb,
             csa, csb, wsa, wsb):
        wid = lax.axis_index("s") * 2 + lax.axis_index("c")
        lo_blk = wid * n_base
        n_blk = jnp.where(wid == _NW - 1, n_last, n_base)
        lane = lax.iota(jnp.int32, _LANES)

        def fetch(blk_off, tb, cs):
            pltpu.async_copy(
                tab_hbm.at[:, pl.ds((lo_blk + blk_off) * _BLK, _BLK)], tb, cs)

        def wait_fetch(tb, cs):
            pltpu.make_async_copy(
                tab_hbm.at[:, pl.ds(0, _BLK)], tb, cs).wait()

        def transpose_block(blk_off, tb, ro, ws, first):
            @pl.when(jnp.logical_not(first))
            def _():
                pltpu.make_async_copy(ro, out_hbm.at[pl.ds(0, _BLK * _DIM)],
                                      ws).wait()
            for c in range(_BLK):
                cvec = jnp.full((_LANES,), c, jnp.int32)
                for k in range(_DIM // _LANES):
                    g = plsc.load_gather(tb, [k * _LANES + lane, cvec])
                    ro[pl.ds(c * _DIM + k * _LANES, _LANES)] = g
            pltpu.async_copy(
                ro,
                out_hbm.at[pl.ds((lo_blk + blk_off) * _BLK * _DIM,
                                 _BLK * _DIM)],
                ws)

        fetch(0, tba, csa)
        fetch(1, tbb, csb)

        def pair_body(p, _):
            ga = 2 * p
            wait_fetch(tba, csa)
            transpose_block(ga, tba, roa, wsa, p == 0)
            @pl.when(2 * p + 2 < n_blk)
            def _():
                fetch(2 * p + 2, tba, csa)
            wait_fetch(tbb, csb)
            transpose_block(ga + 1, tbb, rob, wsb, p == 0)
            @pl.when(2 * p + 3 < n_blk)
            def _():
                fetch(2 * p + 3, tbb, csb)
            return _

        lax.fori_loop(0, n_blk // 2, pair_body, None)
        pltpu.make_async_copy(roa, out_hbm.at[pl.ds(0, _BLK * _DIM)],
                              wsa).wait()
        pltpu.make_async_copy(rob, out_hbm.at[pl.ds(0, _BLK * _DIM)],
                              wsb).wait()

        @pl.when(wid == _NW - 1)
        def _():
            pltpu.sync_copy(tail_hbm, tailb)
            ntail = _V - _VMAIN
            for c in range(ntail):
                cvec = jnp.full((_LANES,), c, jnp.int32)
                for k in range(_DIM // _LANES):
                    g = plsc.load_gather(
                        tailb, [(k * _LANES + lane) * ntail + cvec])
                    roa[pl.ds(c * _DIM + k * _LANES, _LANES)] = g
            pltpu.sync_copy(
                roa.at[pl.ds(0, ntail * _DIM)],
                out_hbm.at[pl.ds(_VMAIN * _DIM, ntail * _DIM)])

    return kern


def _gather_dot_kernel(batch: int):
    b_per_w = batch // _NW
    n_chunks = b_per_w // _IDX_CHUNK
    n_blocks = b_per_w // _LANES
    mesh = plsc.VectorSubcoreMesh(core_axis_name="c", subcore_axis_name="s")

    @functools.partial(
        pl.kernel,
        out_type=(
            jax.ShapeDtypeStruct((batch,), jnp.float32),
            jax.ShapeDtypeStruct((batch,), jnp.float32),
        ),
        mesh=mesh,
        scratch_types=[
            pltpu.VMEM((n_chunks, _IDX_CHUNK), jnp.int32),
            pltpu.VMEM((n_chunks, _IDX_CHUNK), jnp.int32),
            pltpu.VMEM((n_chunks, _IDX_CHUNK), jnp.int32),
            pltpu.VMEM((b_per_w, _DIM), jnp.float32),
            pltpu.VMEM((b_per_w, _DIM), jnp.float32),
            pltpu.VMEM((b_per_w, _DIM), jnp.float32),
            pltpu.VMEM((b_per_w,), jnp.float32),
            pltpu.VMEM((b_per_w,), jnp.float32),
            pltpu.SemaphoreType.DMA,
        ],
        compiler_params=pltpu.CompilerParams(
            needs_layout_passes=False, use_tc_tiling_on_sc=False),
    )
    def kern(user_hbm, item_i_hbm, item_j_hbm, utab_hbm, itab_hbm,
             out_i_hbm, out_j_hbm,
             uidx_v, iidx_v, jidx_v, urows_v, irows_v, jrows_v,
             outi_v, outj_v, sem):
        wid = lax.axis_index("s") * 2 + lax.axis_index("c")
        row0 = wid * n_chunks

        pltpu.sync_copy(user_hbm.at[pl.ds(row0, n_chunks)], uidx_v)
        pltpu.sync_copy(item_i_hbm.at[pl.ds(row0, n_chunks)], iidx_v)
        pltpu.sync_copy(item_j_hbm.at[pl.ds(row0, n_chunks)], jidx_v)

        copies = []
        for k in range(n_chunks):
            dst = pl.ds(k * _IDX_CHUNK, _IDX_CHUNK)
            copies.append(pltpu.async_copy(utab_hbm.at[uidx_v.at[k]],
                                           urows_v.at[dst], sem))
            copies.append(pltpu.async_copy(itab_hbm.at[iidx_v.at[k]],
                                           irows_v.at[dst], sem))
            copies.append(pltpu.async_copy(itab_hbm.at[jidx_v.at[k]],
                                           jrows_v.at[dst], sem))
        for c in copies:
            c.wait()

        lane = lax.iota(jnp.int32, _LANES)

        def body(blk, _):
            row = blk * _LANES + lane
            acc_i = [jnp.zeros((_LANES,), jnp.float32) for _ in range(4)]
            acc_j = [jnp.zeros((_LANES,), jnp.float32) for _ in range(4)]
            for d in range(_DIM):
                col = jnp.full((_LANES,), d, jnp.int32)
                ug = plsc.load_gather(urows_v, [row, col])
                ig = plsc.load_gather(irows_v, [row, col])
                jg = plsc.load_gather(jrows_v, [row, col])
                acc_i[d % 4] = acc_i[d % 4] + ug * ig
                acc_j[d % 4] = acc_j[d % 4] + ug * jg
            outi_v[pl.ds(blk * _LANES, _LANES)] = (
                (acc_i[0] + acc_i[1]) + (acc_i[2] + acc_i[3]))
            outj_v[pl.ds(blk * _LANES, _LANES)] = (
                (acc_j[0] + acc_j[1]) + (acc_j[2] + acc_j[3]))
            return _

        lax.fori_loop(0, n_blocks, body, None)

        base = wid * b_per_w
        pltpu.sync_copy(outi_v, out_i_hbm.at[pl.ds(base, b_per_w)])
        pltpu.sync_copy(outj_v, out_j_hbm.at[pl.ds(base, b_per_w)])

    return kern


@jax.jit
def kernel(user, item_i, item_j, user_emb_weight, item_emb_weight):
    batch = user.shape[0]
    relayout = _relayout_kernel()
    gather_dot = _gather_dot_kernel(batch)

    ut = user_emb_weight.T
    it = item_emb_weight.T
    utail = ut[:, _VMAIN:].reshape(-1)
    itail = it[:, _VMAIN:].reshape(-1)

    u_flat = relayout(ut, utail)
    i_flat = relayout(it, itail)
    u_rows = u_flat.reshape(_V, _DIM)
    i_rows = i_flat.reshape(_V, _DIM)

    n_chunks_total = batch // _IDX_CHUNK
    rating_i, rating_j = gather_dot(
        user.reshape(n_chunks_total, _IDX_CHUNK),
        item_i.reshape(n_chunks_total, _IDX_CHUNK),
        item_j.reshape(n_chunks_total, _IDX_CHUNK),
        u_rows,
        i_rows,
    )
    return (rating_i, rating_j)


# speedup vs baseline: 5.3244x; 5.3244x over previous
import functools

import jax
import jax.numpy as jnp
from jax import lax
from jax.experimental import pallas as pl
from jax.experimental.pallas import tpu as pltpu
from jax.experimental.pallas import tpu_sc as plsc

_DIM = 64
_LANES = 16
_NW = 32
_BLK = 128
_V = 1000000
_VMAIN = 999936
_NBLK_BASE = 244
_NBLK_LAST = 248
_B = 16384
_IDX_CHUNK = 128


def _extract_kernel():
    mesh = plsc.VectorSubcoreMesh(core_axis_name="c", subcore_axis_name="s")
    lane = None

    @functools.partial(
        pl.kernel,
        out_type=(
            jax.ShapeDtypeStruct((_B + 1, _BLK), jnp.float32),
            jax.ShapeDtypeStruct((2 * _B + 1, _BLK), jnp.float32),
        ),
        mesh=mesh,
        scratch_types=[
            pltpu.VMEM((1024,), jnp.int32),
            pltpu.VMEM((_B + 16,), jnp.int32),
            pltpu.VMEM((_B + 16,), jnp.int32),
            pltpu.VMEM((2 * _B + 16,), jnp.int32),
            pltpu.VMEM((2 * _B + 16,), jnp.int32),
            pltpu.VMEM((_DIM, _BLK), jnp.float32),
            pltpu.VMEM((_DIM, _BLK), jnp.float32),
            pltpu.VMEM((_DIM * _DIM,), jnp.float32),
            pltpu.VMEM((2, 16, _BLK), jnp.float32),
            pltpu.VMEM((2, 16), jnp.int32),
            pltpu.VMEM((16,), jnp.int32),
            pltpu.VMEM((16,), jnp.int32),
            pltpu.SemaphoreType.DMA,
            pltpu.SemaphoreType.DMA,
            pltpu.SemaphoreType.DMA,
        ],
        compiler_params=pltpu.CompilerParams(
            needs_layout_passes=False, use_tc_tiling_on_sc=True),
    )
    def kern(u_hbm, ii_hbm, ij_hbm, ut_hbm, it_hbm, utail_hbm, itail_hbm,
             usc_hbm, isc_hbm,
             ibuf, hu, bu, hi, bi, tba, tbb, tailb, rowg, slotg,
             bndu, bndi, csa, csb, ssem):
        wid = lax.axis_index("s") * 2 + lax.axis_index("c")
        lane = lax.iota(jnp.int32, _LANES)
        lo = wid * _NBLK_BASE * _BLK
        n_blk = jnp.where(wid == _NW - 1, _NBLK_LAST, _NBLK_BASE)
        span = jnp.where(wid == _NW - 1, _V - lo, _NBLK_BASE * _BLK)

        def prefilter(src_hbm, slot_base, dst, cap, cnt0):
            def piece(p, cnt):
                pltpu.sync_copy(src_hbm.at[pl.ds(p * 1024, 1024)], ibuf)
                def vec(q, cnt):
                    v = ibuf[pl.ds(q * 16, 16)]
                    vloc = v - lo
                    m = (vloc >= 0) & (vloc < span)
                    mi = m.astype(jnp.int32)
                    cs = plsc.cumsum(mi)
                    slot = slot_base + p * 1024 + q * 16 + lane
                    h = (vloc << 15) | slot
                    tgt = jnp.where(m, cnt + cs - mi, cap)
                    plsc.store_scatter(dst, [tgt], h)
                    return cnt + cs[15]
                return lax.fori_loop(0, 64, vec, cnt)
            return lax.fori_loop(0, _B // 1024, piece, cnt0)

        cnt_u = prefilter(u_hbm, 0, hu, _B, jnp.int32(0))
        cnt_i = prefilter(ii_hbm, 0, hi, 2 * _B, jnp.int32(0))
        cnt_i = prefilter(ij_hbm, _B, hi, 2 * _B, cnt_i)

        def bucket(src, dst, cap, cnt, bnd_ref):
            nhv = (cnt + 15) >> 4
            def rpass(r, carry):
                cnt2, bnd = carry
                def vec(t, cnt2):
                    h = src[pl.ds(t * 16, 16)]
                    valid = (t * 16 + lane) < cnt
                    m = valid & ((h >> 26) == r)
                    mi = m.astype(jnp.int32)
                    cs = plsc.cumsum(mi)
                    tgt = jnp.where(m, cnt2 + cs - mi, cap)
                    plsc.store_scatter(dst, [tgt], h)
                    return cnt2 + cs[15]
                cnt2 = lax.fori_loop(0, nhv, vec, cnt2)
                bnd = jnp.where(lane == r, cnt2, bnd)
                return (cnt2, bnd)
            _, bnd = lax.fori_loop(0, 16, rpass,
                                   (jnp.int32(0), jnp.zeros((16,), jnp.int32)))
            bnd_ref[pl.ds(0, 16)] = bnd
        bucket(hu, bu, _B, cnt_u, bndu)
        bucket(hi, bi, 2 * _B, cnt_i, bndi)

        dump_u = jnp.int32(_B)
        dump_i = jnp.int32(2 * _B)

        def run_pass(tab_hbm, hits, cnt, bnd_ref, out_hbm, dump):
            def fetch(g, tb, cs):
                pltpu.async_copy(
                    tab_hbm.at[:, pl.ds(lo + g * _BLK, _BLK)], tb, cs)

            def wait_fetch(tb, cs):
                pltpu.make_async_copy(
                    tab_hbm.at[:, pl.ds(0, _BLK)], tb, cs).wait()

            def append_hits(g_off, hcnt, gather_col):
                r = g_off >> 4
                t1v = plsc.load_gather(bnd_ref, [jnp.full((16,), r, jnp.int32)])
                t0v = plsc.load_gather(
                    bnd_ref, [jnp.full((16,), jnp.maximum(r - 1, 0), jnp.int32)])
                t0 = jnp.where(r == 0, 0, t0v[0]) >> 4
                t1 = (t1v[0] + 15) >> 4

                def hv_body(t, hcnt):
                    h = hits[pl.ds(t * 16, 16)]
                    valid = (t * 16 + lane) < cnt
                    vloc = h >> 15
                    m = valid & ((vloc >> 7) == g_off)
                    mi = m.astype(jnp.int32)
                    cs = plsc.cumsum(mi)
                    pc = cs[15]
                    @pl.when(pc > 0)
                    def _():
                        for j in range(16):
                            @pl.when(mi[j] > 0)
                            def _():
                                pos = hcnt + cs[j] - 1
                                prow = (pos >> 4) & 1
                                pslot = pos & 15
                                cloc = vloc[j] & 127
                                slot = h[j] & 32767
                                @pl.when(pslot == 0)
                                def _():
                                    @pl.when(pos >= 32)
                                    def _():
                                        pltpu.make_async_copy(
                                            rowg.at[0],
                                            out_hbm.at[slotg.at[0]],
                                            ssem).wait()
                                    plsc.store_scatter(
                                        slotg,
                                        [jnp.full((16,), prow, jnp.int32),
                                         lane],
                                        jnp.full((16,), dump, jnp.int32))
                                gather_col(cloc, prow, pslot)
                                plsc.store_scatter(
                                    slotg,
                                    [jnp.full((16,), prow, jnp.int32),
                                     jnp.full((16,), pslot, jnp.int32)],
                                    jnp.full((16,), slot, jnp.int32))
                                @pl.when(pslot == 15)
                                def _():
                                    pltpu.async_copy(
                                        rowg.at[prow],
                                        out_hbm.at[slotg.at[prow]], ssem)
                    return hcnt + pc
                return lax.fori_loop(t0, t1, hv_body, hcnt)

            def mk_gather(tb):
                def gather_col(cloc, prow, pslot):
                    cvec = jnp.full((_LANES,), cloc, jnp.int32)
                    pv = jnp.full((_LANES,), prow, jnp.int32)
                    sv = jnp.full((_LANES,), pslot, jnp.int32)
                    for k in range(_DIM // _LANES):
                        g = plsc.load_gather(tb, [k * _LANES + lane, cvec])
                        plsc.store_scatter(rowg, [pv, sv, k * _LANES + lane], g)
                return gather_col

            fetch(0, tba, csa)
            fetch(1, tbb, csb)

            def pair(p, hcnt):
                g0 = 2 * p
                wait_fetch(tba, csa)
                hcnt = append_hits(g0, hcnt, mk_gather(tba))
                @pl.when(g0 + 2 < n_blk)
                def _():
                    fetch(g0 + 2, tba, csa)
                wait_fetch(tbb, csb)
                hcnt = append_hits(g0 + 1, hcnt, mk_gather(tbb))
                @pl.when(g0 + 3 < n_blk)
                def _():
                    fetch(g0 + 3, tbb, csb)
                return hcnt

            hcnt = lax.fori_loop(0, n_blk // 2, pair, jnp.int32(0))

            ntail = _V - _VMAIN

            def tail_gather(cloc, prow, pslot):
                pv = jnp.full((_LANES,), prow, jnp.int32)
                sv = jnp.full((_LANES,), pslot, jnp.int32)
                for k in range(_DIM // _LANES):
                    idx = (k * _LANES + lane) * ntail + cloc
                    g = plsc.load_gather(tailb, [idx])
                    plsc.store_scatter(rowg, [pv, sv, k * _LANES + lane], g)

            g_tail = jnp.where(wid == _NW - 1, jnp.int32(_NBLK_LAST),
                               jnp.int32(_NBLK_LAST + 1))
            hcnt = append_hits(g_tail, hcnt, tail_gather)

            @pl.when((hcnt & 15) != 0)
            def _():
                prow = (hcnt >> 4) & 1
                pltpu.async_copy(rowg.at[prow], out_hbm.at[slotg.at[prow]],
                                 ssem)
            @pl.when(hcnt > 0)
            def _():
                pltpu.make_async_copy(
                    rowg.at[0], out_hbm.at[slotg.at[0]], ssem).wait()
            @pl.when(hcnt > 16)
            def _():
                pltpu.make_async_copy(
                    rowg.at[0], out_hbm.at[slotg.at[0]], ssem).wait()

        pltpu.sync_copy(utail_hbm, tailb)
        run_pass(ut_hbm, bu, cnt_u, bndu, usc_hbm, dump_u)
        pltpu.sync_copy(itail_hbm, tailb)
        run_pass(it_hbm, bi, cnt_i, bndi, isc_hbm, dump_i)

    return kern


def _gather_dot_kernel():
    b_per_w = _B // _NW
    n_rounds = b_per_w // _IDX_CHUNK
    mesh = plsc.VectorSubcoreMesh(core_axis_name="c", subcore_axis_name="s")

    @functools.partial(
        pl.kernel,
        out_type=(
            jax.ShapeDtypeStruct((_B,), jnp.float32),
            jax.ShapeDtypeStruct((_B,), jnp.float32),
        ),
        mesh=mesh,
        scratch_types=[
            pltpu.VMEM((_IDX_CHUNK, _BLK), jnp.float32),
            pltpu.VMEM((_IDX_CHUNK, _BLK), jnp.float32),
            pltpu.VMEM((_IDX_CHUNK, _BLK), jnp.float32),
            pltpu.VMEM((b_per_w,), jnp.float32),
            pltpu.VMEM((b_per_w,), jnp.float32),
            pltpu.SemaphoreType.DMA,
        ],
        compiler_params=pltpu.CompilerParams(
            needs_layout_passes=False, use_tc_tiling_on_sc=False),
    )
    def kern(usc_hbm, isc_hbm, out_i_hbm, out_j_hbm,
             urows, irows, jrows, outi, outj, sem):
        wid = lax.axis_index("s") * 2 + lax.axis_index("c")
        base = wid * b_per_w
        lane = lax.iota(jnp.int32, _LANES)

        for rnd in range(n_rounds):
            r0 = base + rnd * _IDX_CHUNK
            cu = pltpu.async_copy(usc_hbm.at[pl.ds(r0, _IDX_CHUNK)],
                                  urows, sem)
            ci = pltpu.async_copy(isc_hbm.at[pl.ds(r0, _IDX_CHUNK)],
                                  irows, sem)
            cj = pltpu.async_copy(isc_hbm.at[pl.ds(_B + r0, _IDX_CHUNK)],
                                  jrows, sem)
            cu.wait(); ci.wait(); cj.wait()

            def body(blk, _):
                row = blk * _LANES + lane
                acc_i = [jnp.zeros((_LANES,), jnp.float32) for _ in range(4)]
                acc_j = [jnp.zeros((_LANES,), jnp.float32) for _ in range(4)]
                for d in range(_DIM):
                    col = jnp.full((_LANES,), d, jnp.int32)
                    ug = plsc.load_gather(urows, [row, col])
                    ig = plsc.load_gather(irows, [row, col])
                    jg = plsc.load_gather(jrows, [row, col])
                    acc_i[d % 4] = acc_i[d % 4] + ug * ig
                    acc_j[d % 4] = acc_j[d % 4] + ug * jg
                o = pl.ds(rnd * _IDX_CHUNK + blk * _LANES, _LANES)
                outi[o] = (acc_i[0] + acc_i[1]) + (acc_i[2] + acc_i[3])
                outj[o] = (acc_j[0] + acc_j[1]) + (acc_j[2] + acc_j[3])
                return _
            lax.fori_loop(0, _IDX_CHUNK // _LANES, body, None)

        pltpu.sync_copy(outi, out_i_hbm.at[pl.ds(base, b_per_w)])
        pltpu.sync_copy(outj, out_j_hbm.at[pl.ds(base, b_per_w)])

    return kern


@jax.jit
def kernel(user, item_i, item_j, user_emb_weight, item_emb_weight):
    extract = _extract_kernel()
    gather_dot = _gather_dot_kernel()

    ut = user_emb_weight.T
    it = item_emb_weight.T
    utail = ut[:, _VMAIN:].reshape(-1)
    itail = it[:, _VMAIN:].reshape(-1)

    usc, isc = extract(user, item_i, item_j, ut, it, utail, itail)
    rating_i, rating_j = gather_dot(usc, isc)
    return (rating_i, rating_j)


# speedup vs baseline: 5.3320x; 1.0014x over previous
import functools

import jax
import jax.numpy as jnp
from jax import lax
from jax.experimental import pallas as pl
from jax.experimental.pallas import tpu as pltpu
from jax.experimental.pallas import tpu_sc as plsc

_DIM = 64
_LANES = 16
_NW = 32
_BLK = 128
_V = 1000000
_VMAIN = 999936
_NBLK_BASE = 244
_NBLK_LAST = 248
_B = 16384
_IDX_CHUNK = 128


def _extract_kernel():
    mesh = plsc.VectorSubcoreMesh(core_axis_name="c", subcore_axis_name="s")
    lane = None

    @functools.partial(
        pl.kernel,
        out_type=(
            jax.ShapeDtypeStruct((_B + 1, _BLK), jnp.float32),
            jax.ShapeDtypeStruct((2 * _B + 1, _BLK), jnp.float32),
        ),
        mesh=mesh,
        scratch_types=[
            pltpu.VMEM((1024,), jnp.int32),
            pltpu.VMEM((_B + 16,), jnp.int32),
            pltpu.VMEM((_B + 16,), jnp.int32),
            pltpu.VMEM((2 * _B + 16,), jnp.int32),
            pltpu.VMEM((2 * _B + 16,), jnp.int32),
            pltpu.VMEM((_DIM, _BLK), jnp.float32),
            pltpu.VMEM((_DIM, _BLK), jnp.float32),
            pltpu.VMEM((_DIM * _DIM,), jnp.float32),
            pltpu.VMEM((4, 16, _BLK), jnp.float32),
            pltpu.VMEM((4, 16), jnp.int32),
            pltpu.VMEM((16,), jnp.int32),
            pltpu.VMEM((16,), jnp.int32),
            pltpu.SemaphoreType.DMA,
            pltpu.SemaphoreType.DMA,
            pltpu.SemaphoreType.DMA,
        ],
        compiler_params=pltpu.CompilerParams(
            needs_layout_passes=False, use_tc_tiling_on_sc=True),
    )
    def kern(u_hbm, ii_hbm, ij_hbm, ut_hbm, it_hbm, utail_hbm, itail_hbm,
             usc_hbm, isc_hbm,
             ibuf, hu, bu, hi, bi, tba, tbb, tailb, rowg, slotg,
             bndu, bndi, csa, csb, ssem):
        wid = lax.axis_index("s") * 2 + lax.axis_index("c")
        lane = lax.iota(jnp.int32, _LANES)
        lo = wid * _NBLK_BASE * _BLK
        n_blk = jnp.where(wid == _NW - 1, _NBLK_LAST, _NBLK_BASE)
        span = jnp.where(wid == _NW - 1, _V - lo, _NBLK_BASE * _BLK)

        def prefilter(src_hbm, slot_base, dst, cap, cnt0):
            def piece(p, cnt):
                pltpu.sync_copy(src_hbm.at[pl.ds(p * 1024, 1024)], ibuf)
                def vec(q, cnt):
                    v = ibuf[pl.ds(q * 16, 16)]
                    vloc = v - lo
                    m = (vloc >= 0) & (vloc < span)
                    mi = m.astype(jnp.int32)
                    cs = plsc.cumsum(mi)
                    slot = slot_base + p * 1024 + q * 16 + lane
                    h = (vloc << 15) | slot
                    tgt = jnp.where(m, cnt + cs - mi, cap)
                    plsc.store_scatter(dst, [tgt], h)
                    return cnt + cs[15]
                return lax.fori_loop(0, 64, vec, cnt)
            return lax.fori_loop(0, _B // 1024, piece, cnt0)

        cnt_u = prefilter(u_hbm, 0, hu, _B, jnp.int32(0))
        cnt_i = prefilter(ii_hbm, 0, hi, 2 * _B, jnp.int32(0))
        cnt_i = prefilter(ij_hbm, _B, hi, 2 * _B, cnt_i)

        def bucket(src, dst, cap, cnt, bnd_ref):
            nhv = (cnt + 15) >> 4
            def rpass(r, carry):
                cnt2, bnd = carry
                def vec(t, cnt2):
                    h = src[pl.ds(t * 16, 16)]
                    valid = (t * 16 + lane) < cnt
                    m = valid & ((h >> 26) == r)
                    mi = m.astype(jnp.int32)
                    cs = plsc.cumsum(mi)
                    tgt = jnp.where(m, cnt2 + cs - mi, cap)
                    plsc.store_scatter(dst, [tgt], h)
                    return cnt2 + cs[15]
                cnt2 = lax.fori_loop(0, nhv, vec, cnt2)
                bnd = jnp.where(lane == r, cnt2, bnd)
                return (cnt2, bnd)
            _, bnd = lax.fori_loop(0, 16, rpass,
                                   (jnp.int32(0), jnp.zeros((16,), jnp.int32)))
            bnd_ref[pl.ds(0, 16)] = bnd
        bucket(hu, bu, _B, cnt_u, bndu)
        bucket(hi, bi, 2 * _B, cnt_i, bndi)

        dump_u = jnp.int32(_B)
        dump_i = jnp.int32(2 * _B)

        def run_pass(tab_hbm, hits, cnt, bnd_ref, out_hbm, dump):
            def fetch(g, tb, cs):
                pltpu.async_copy(
                    tab_hbm.at[:, pl.ds(lo + g * _BLK, _BLK)], tb, cs)

            def wait_fetch(tb, cs):
                pltpu.make_async_copy(
                    tab_hbm.at[:, pl.ds(0, _BLK)], tb, cs).wait()

            def append_hits(g_off, hcnt, gather_col):
                r = g_off >> 4
                t1v = plsc.load_gather(bnd_ref, [jnp.full((16,), r, jnp.int32)])
                t0v = plsc.load_gather(
                    bnd_ref, [jnp.full((16,), jnp.maximum(r - 1, 0), jnp.int32)])
                t0 = jnp.where(r == 0, 0, t0v[0]) >> 4
                t1 = (t1v[0] + 15) >> 4

                def hv_body(t, hcnt):
                    h = hits[pl.ds(t * 16, 16)]
                    valid = (t * 16 + lane) < cnt
                    vloc = h >> 15
                    m = valid & ((vloc >> 7) == g_off)
                    mi = m.astype(jnp.int32)
                    cs = plsc.cumsum(mi)
                    pc = cs[15]
                    @pl.when(pc > 0)
                    def _():
                        for j in range(16):
                            @pl.when(mi[j] > 0)
                            def _():
                                pos = hcnt + cs[j] - 1
                                prow = (pos >> 4) & 3
                                pslot = pos & 15
                                cloc = vloc[j] & 127
                                slot = h[j] & 32767
                                @pl.when(pslot == 0)
                                def _():
                                    @pl.when(pos >= 64)
                                    def _():
                                        pltpu.make_async_copy(
                                            rowg.at[0],
                                            out_hbm.at[slotg.at[0]],
                                            ssem).wait()
                                    plsc.store_scatter(
                                        slotg,
                                        [jnp.full((16,), prow, jnp.int32),
                                         lane],
                                        jnp.full((16,), dump, jnp.int32))
                                gather_col(cloc, prow, pslot)
                                plsc.store_scatter(
                                    slotg,
                                    [jnp.full((16,), prow, jnp.int32),
                                     jnp.full((16,), pslot, jnp.int32)],
                                    jnp.full((16,), slot, jnp.int32))
                                @pl.when(pslot == 15)
                                def _():
                                    pltpu.async_copy(
                                        rowg.at[prow],
                                        out_hbm.at[slotg.at[prow]], ssem)
                    return hcnt + pc
                return lax.fori_loop(t0, t1, hv_body, hcnt)

            def mk_gather(tb):
                def gather_col(cloc, prow, pslot):
                    cvec = jnp.full((_LANES,), cloc, jnp.int32)
                    pv = jnp.full((_LANES,), prow, jnp.int32)
                    sv = jnp.full((_LANES,), pslot, jnp.int32)
                    for k in range(_DIM // _LANES):
                        g = plsc.load_gather(tb, [k * _LANES + lane, cvec])
                        plsc.store_scatter(rowg, [pv, sv, k * _LANES + lane], g)
                return gather_col

            fetch(0, tba, csa)
            fetch(1, tbb, csb)

            def pair(p, hcnt):
                g0 = 2 * p
                wait_fetch(tba, csa)
                hcnt = append_hits(g0, hcnt, mk_gather(tba))
                @pl.when(g0 + 2 < n_blk)
                def _():
                    fetch(g0 + 2, tba, csa)
                wait_fetch(tbb, csb)
                hcnt = append_hits(g0 + 1, hcnt, mk_gather(tbb))
                @pl.when(g0 + 3 < n_blk)
                def _():
                    fetch(g0 + 3, tbb, csb)
                return hcnt

            hcnt = lax.fori_loop(0, n_blk // 2, pair, jnp.int32(0))

            ntail = _V - _VMAIN

            def tail_gather(cloc, prow, pslot):
                pv = jnp.full((_LANES,), prow, jnp.int32)
                sv = jnp.full((_LANES,), pslot, jnp.int32)
                for k in range(_DIM // _LANES):
                    idx = (k * _LANES + lane) * ntail + cloc
                    g = plsc.load_gather(tailb, [idx])
                    plsc.store_scatter(rowg, [pv, sv, k * _LANES + lane], g)

            g_tail = jnp.where(wid == _NW - 1, jnp.int32(_NBLK_LAST),
                               jnp.int32(_NBLK_LAST + 1))
            hcnt = append_hits(g_tail, hcnt, tail_gather)

            @pl.when((hcnt & 15) != 0)
            def _():
                prow = (hcnt >> 4) & 3
                pltpu.async_copy(rowg.at[prow], out_hbm.at[slotg.at[prow]],
                                 ssem)
            for thresh in (0, 16, 32, 48):
                @pl.when(hcnt > thresh)
                def _():
                    pltpu.make_async_copy(
                        rowg.at[0], out_hbm.at[slotg.at[0]], ssem).wait()

        pltpu.sync_copy(utail_hbm, tailb)
        run_pass(ut_hbm, bu, cnt_u, bndu, usc_hbm, dump_u)
        pltpu.sync_copy(itail_hbm, tailb)
        run_pass(it_hbm, bi, cnt_i, bndi, isc_hbm, dump_i)

    return kern


def _gather_dot_kernel():
    b_per_w = _B // _NW
    n_rounds = b_per_w // _IDX_CHUNK
    mesh = plsc.VectorSubcoreMesh(core_axis_name="c", subcore_axis_name="s")

    @functools.partial(
        pl.kernel,
        out_type=(
            jax.ShapeDtypeStruct((_B,), jnp.float32),
            jax.ShapeDtypeStruct((_B,), jnp.float32),
        ),
        mesh=mesh,
        scratch_types=[
            pltpu.VMEM((_IDX_CHUNK, _BLK), jnp.float32),
            pltpu.VMEM((_IDX_CHUNK, _BLK), jnp.float32),
            pltpu.VMEM((_IDX_CHUNK, _BLK), jnp.float32),
            pltpu.VMEM((b_per_w,), jnp.float32),
            pltpu.VMEM((b_per_w,), jnp.float32),
            pltpu.SemaphoreType.DMA,
        ],
        compiler_params=pltpu.CompilerParams(
            needs_layout_passes=False, use_tc_tiling_on_sc=False),
    )
    def kern(usc_hbm, isc_hbm, out_i_hbm, out_j_hbm,
             urows, irows, jrows, outi, outj, sem):
        wid = lax.axis_index("s") * 2 + lax.axis_index("c")
        base = wid * b_per_w
        lane = lax.iota(jnp.int32, _LANES)

        for rnd in range(n_rounds):
            r0 = base + rnd * _IDX_CHUNK
            cu = pltpu.async_copy(usc_hbm.at[pl.ds(r0, _IDX_CHUNK)],
                                  urows, sem)
            ci = pltpu.async_copy(isc_hbm.at[pl.ds(r0, _IDX_CHUNK)],
                                  irows, sem)
            cj = pltpu.async_copy(isc_hbm.at[pl.ds(_B + r0, _IDX_CHUNK)],
                                  jrows, sem)
            cu.wait(); ci.wait(); cj.wait()

            def body(blk, _):
                row = blk * _LANES + lane
                acc_i = [jnp.zeros((_LANES,), jnp.float32) for _ in range(4)]
                acc_j = [jnp.zeros((_LANES,), jnp.float32) for _ in range(4)]
                for d in range(_DIM):
                    col = jnp.full((_LANES,), d, jnp.int32)
                    ug = plsc.load_gather(urows, [row, col])
                    ig = plsc.load_gather(irows, [row, col])
                    jg = plsc.load_gather(jrows, [row, col])
                    acc_i[d % 4] = acc_i[d % 4] + ug * ig
                    acc_j[d % 4] = acc_j[d % 4] + ug * jg
                o = pl.ds(rnd * _IDX_CHUNK + blk * _LANES, _LANES)
                outi[o] = (acc_i[0] + acc_i[1]) + (acc_i[2] + acc_i[3])
                outj[o] = (acc_j[0] + acc_j[1]) + (acc_j[2] + acc_j[3])
                return _
            lax.fori_loop(0, _IDX_CHUNK // _LANES, body, None)

        pltpu.sync_copy(outi, out_i_hbm.at[pl.ds(base, b_per_w)])
        pltpu.sync_copy(outj, out_j_hbm.at[pl.ds(base, b_per_w)])

    return kern


@jax.jit
def kernel(user, item_i, item_j, user_emb_weight, item_emb_weight):
    extract = _extract_kernel()
    gather_dot = _gather_dot_kernel()

    ut = user_emb_weight.T
    it = item_emb_weight.T
    utail = ut[:, _VMAIN:].reshape(-1)
    itail = it[:, _VMAIN:].reshape(-1)

    usc, isc = extract(user, item_i, item_j, ut, it, utail, itail)
    rating_i, rating_j = gather_dot(usc, isc)
    return (rating_i, rating_j)


# speedup vs baseline: 5.3664x; 1.0064x over previous
import functools

import jax
import jax.numpy as jnp
from jax import lax
from jax.experimental import pallas as pl
from jax.experimental.pallas import tpu as pltpu
from jax.experimental.pallas import tpu_sc as plsc

_DIM = 64
_LANES = 16
_NW = 32
_BLK = 128
_V = 1000000
_VMAIN = 999936
_NBLK_BASE = 244
_NBLK_LAST = 248
_B = 16384
_IDX_CHUNK = 128


def _extract_kernel():
    mesh = plsc.VectorSubcoreMesh(core_axis_name="c", subcore_axis_name="s")
    lane = None

    @functools.partial(
        pl.kernel,
        out_type=(
            jax.ShapeDtypeStruct((_B + 1, _BLK), jnp.float32),
            jax.ShapeDtypeStruct((2 * _B + 1, _BLK), jnp.float32),
        ),
        mesh=mesh,
        scratch_types=[
            pltpu.VMEM((1024,), jnp.int32),
            pltpu.VMEM((_B + 16,), jnp.int32),
            pltpu.VMEM((_B + 16,), jnp.int32),
            pltpu.VMEM((2 * _B + 16,), jnp.int32),
            pltpu.VMEM((2 * _B + 16,), jnp.int32),
            pltpu.VMEM((_DIM, _BLK), jnp.float32),
            pltpu.VMEM((_DIM, _BLK), jnp.float32),
            pltpu.VMEM((_DIM * _DIM,), jnp.float32),
            pltpu.VMEM((4, 16, _BLK), jnp.float32),
            pltpu.VMEM((4, 16), jnp.int32),
            pltpu.VMEM((16,), jnp.int32),
            pltpu.VMEM((16,), jnp.int32),
            pltpu.SemaphoreType.DMA,
            pltpu.SemaphoreType.DMA,
            pltpu.SemaphoreType.DMA,
        ],
        compiler_params=pltpu.CompilerParams(
            needs_layout_passes=False, use_tc_tiling_on_sc=True),
    )
    def kern(u_hbm, ii_hbm, ij_hbm, ut_hbm, it_hbm, utail_hbm, itail_hbm,
             usc_hbm, isc_hbm,
             ibuf, hu, bu, hi, bi, tba, tbb, tailb, rowg, slotg,
             bndu, bndi, csa, csb, ssem):
        wid = lax.axis_index("s") * 2 + lax.axis_index("c")
        lane = lax.iota(jnp.int32, _LANES)
        lo = wid * _NBLK_BASE * _BLK
        n_blk = jnp.where(wid == _NW - 1, _NBLK_LAST, _NBLK_BASE)
        span = jnp.where(wid == _NW - 1, _V - lo, _NBLK_BASE * _BLK)

        def prefilter(src_hbm, slot_base, dst, cap, cnt0):
            def piece(p, cnt):
                pltpu.sync_copy(src_hbm.at[pl.ds(p * 1024, 1024)], ibuf)
                def vec(q, cnt):
                    v = ibuf[pl.ds(q * 16, 16)]
                    vloc = v - lo
                    m = (vloc >= 0) & (vloc < span)
                    mi = m.astype(jnp.int32)
                    cs = plsc.cumsum(mi)
                    slot = slot_base + p * 1024 + q * 16 + lane
                    h = (vloc << 15) | slot
                    tgt = jnp.where(m, cnt + cs - mi, cap)
                    plsc.store_scatter(dst, [tgt], h)
                    return cnt + cs[15]
                return lax.fori_loop(0, 64, vec, cnt)
            return lax.fori_loop(0, _B // 1024, piece, cnt0)

        cnt_u = prefilter(u_hbm, 0, hu, _B, jnp.int32(0))
        cnt_i = prefilter(ii_hbm, 0, hi, 2 * _B, jnp.int32(0))
        cnt_i = prefilter(ij_hbm, _B, hi, 2 * _B, cnt_i)

        def bucket(src, dst, cap, cnt, bnd_ref):
            nhv = (cnt + 15) >> 4
            def rpass(r, carry):
                cnt2, bnd = carry
                def vec(t, cnt2):
                    h = src[pl.ds(t * 16, 16)]
                    valid = (t * 16 + lane) < cnt
                    m = valid & ((h >> 26) == r)
                    mi = m.astype(jnp.int32)
                    cs = plsc.cumsum(mi)
                    tgt = jnp.where(m, cnt2 + cs - mi, cap)
                    plsc.store_scatter(dst, [tgt], h)
                    return cnt2 + cs[15]
                cnt2 = lax.fori_loop(0, nhv, vec, cnt2)
                bnd = jnp.where(lane == r, cnt2, bnd)
                return (cnt2, bnd)
            _, bnd = lax.fori_loop(0, 16, rpass,
                                   (jnp.int32(0), jnp.zeros((16,), jnp.int32)))
            bnd_ref[pl.ds(0, 16)] = bnd
        bucket(hu, bu, _B, cnt_u, bndu)
        bucket(hi, bi, 2 * _B, cnt_i, bndi)

        dump_u = jnp.int32(_B)
        dump_i = jnp.int32(2 * _B)

        def run_pass(tab_hbm, hits, cnt, bnd_ref, out_hbm, dump):
            def fetch(g, tb, cs):
                pltpu.async_copy(
                    tab_hbm.at[:, pl.ds(lo + g * _BLK, _BLK)], tb, cs)

            def wait_fetch(tb, cs):
                pltpu.make_async_copy(
                    tab_hbm.at[:, pl.ds(0, _BLK)], tb, cs).wait()

            def append_hits(g_off, hcnt, gather_col):
                r = g_off >> 4
                t1v = plsc.load_gather(bnd_ref, [jnp.full((16,), r, jnp.int32)])
                t0v = plsc.load_gather(
                    bnd_ref, [jnp.full((16,), jnp.maximum(r - 1, 0), jnp.int32)])
                t0 = jnp.where(r == 0, 0, t0v[0]) >> 4
                t1 = (t1v[0] + 15) >> 4

                def hv_body(t, hcnt):
                    h = hits[pl.ds(t * 16, 16)]
                    valid = (t * 16 + lane) < cnt
                    vloc = h >> 15
                    m = valid & ((vloc >> 7) == g_off)
                    mi = m.astype(jnp.int32)
                    cs = plsc.cumsum(mi)
                    pc = cs[15]
                    @pl.when(pc > 0)
                    def _():
                        for j in range(16):
                            @pl.when(mi[j] > 0)
                            def _():
                                pos = hcnt + cs[j] - 1
                                prow = (pos >> 4) & 3
                                pslot = pos & 15
                                cloc = vloc[j] & 127
                                slot = h[j] & 32767
                                @pl.when(pslot == 0)
                                def _():
                                    @pl.when(pos >= 64)
                                    def _():
                                        pltpu.make_async_copy(
                                            rowg.at[0],
                                            out_hbm.at[slotg.at[0]],
                                            ssem).wait()
                                    plsc.store_scatter(
                                        slotg,
                                        [jnp.full((16,), prow, jnp.int32),
                                         lane],
                                        jnp.full((16,), dump, jnp.int32))
                                gather_col(cloc, prow, pslot)
                                plsc.store_scatter(
                                    slotg,
                                    [jnp.full((16,), prow, jnp.int32),
                                     jnp.full((16,), pslot, jnp.int32)],
                                    jnp.full((16,), slot, jnp.int32))
                                @pl.when(pslot == 15)
                                def _():
                                    pltpu.async_copy(
                                        rowg.at[prow],
                                        out_hbm.at[slotg.at[prow]], ssem)
                    return hcnt + pc
                return lax.fori_loop(t0, t1, hv_body, hcnt)

            def mk_gather(tb):
                def gather_col(cloc, prow, pslot):
                    cvec = jnp.full((_LANES,), cloc, jnp.int32)
                    pv = jnp.full((_LANES,), prow, jnp.int32)
                    sv = jnp.full((_LANES,), pslot, jnp.int32)
                    for k in range(_DIM // _LANES):
                        g = plsc.load_gather(tb, [k * _LANES + lane, cvec])
                        plsc.store_scatter(rowg, [pv, sv, k * _LANES + lane], g)
                return gather_col

            fetch(0, tba, csa)
            fetch(1, tbb, csb)

            def pair(p, hcnt):
                g0 = 2 * p
                wait_fetch(tba, csa)
                hcnt = append_hits(g0, hcnt, mk_gather(tba))
                @pl.when(g0 + 2 < n_blk)
                def _():
                    fetch(g0 + 2, tba, csa)
                wait_fetch(tbb, csb)
                hcnt = append_hits(g0 + 1, hcnt, mk_gather(tbb))
                @pl.when(g0 + 3 < n_blk)
                def _():
                    fetch(g0 + 3, tbb, csb)
                return hcnt

            hcnt = lax.fori_loop(0, n_blk // 2, pair, jnp.int32(0))

            ntail = _V - _VMAIN

            def tail_gather(cloc, prow, pslot):
                pv = jnp.full((_LANES,), prow, jnp.int32)
                sv = jnp.full((_LANES,), pslot, jnp.int32)
                for k in range(_DIM // _LANES):
                    idx = (k * _LANES + lane) * ntail + cloc
                    g = plsc.load_gather(tailb, [idx])
                    plsc.store_scatter(rowg, [pv, sv, k * _LANES + lane], g)

            g_tail = jnp.where(wid == _NW - 1, jnp.int32(_NBLK_LAST),
                               jnp.int32(_NBLK_LAST + 1))
            hcnt = append_hits(g_tail, hcnt, tail_gather)

            @pl.when((hcnt & 15) != 0)
            def _():
                prow = (hcnt >> 4) & 3
                pltpu.async_copy(rowg.at[prow], out_hbm.at[slotg.at[prow]],
                                 ssem)
            for thresh in (0, 16, 32, 48):
                @pl.when(hcnt > thresh)
                def _():
                    pltpu.make_async_copy(
                        rowg.at[0], out_hbm.at[slotg.at[0]], ssem).wait()

        pltpu.sync_copy(utail_hbm, tailb)
        run_pass(ut_hbm, bu, cnt_u, bndu, usc_hbm, dump_u)
        pltpu.sync_copy(itail_hbm, tailb)
        run_pass(it_hbm, bi, cnt_i, bndi, isc_hbm, dump_i)

    return kern


def _gather_dot_kernel():
    b_per_w = _B // _NW
    n_rounds = b_per_w // _IDX_CHUNK
    mesh = plsc.VectorSubcoreMesh(core_axis_name="c", subcore_axis_name="s")

    @functools.partial(
        pl.kernel,
        out_type=(
            jax.ShapeDtypeStruct((_B,), jnp.float32),
            jax.ShapeDtypeStruct((_B,), jnp.float32),
        ),
        mesh=mesh,
        scratch_types=[
            [pltpu.VMEM((_IDX_CHUNK, _BLK), jnp.float32)] * 2,
            [pltpu.VMEM((_IDX_CHUNK, _BLK), jnp.float32)] * 2,
            [pltpu.VMEM((_IDX_CHUNK, _BLK), jnp.float32)] * 2,
            pltpu.VMEM((b_per_w,), jnp.float32),
            pltpu.VMEM((b_per_w,), jnp.float32),
            [pltpu.SemaphoreType.DMA] * 2,
        ],
        compiler_params=pltpu.CompilerParams(
            needs_layout_passes=False, use_tc_tiling_on_sc=False),
    )
    def kern(usc_hbm, isc_hbm, out_i_hbm, out_j_hbm,
             urows2, irows2, jrows2, outi, outj, sems):
        wid = lax.axis_index("s") * 2 + lax.axis_index("c")
        base = wid * b_per_w
        lane = lax.iota(jnp.int32, _LANES)

        def fire(rnd):
            p = rnd % 2
            r0 = base + rnd * _IDX_CHUNK
            return (
                pltpu.async_copy(usc_hbm.at[pl.ds(r0, _IDX_CHUNK)],
                                 urows2[p], sems[p]),
                pltpu.async_copy(isc_hbm.at[pl.ds(r0, _IDX_CHUNK)],
                                 irows2[p], sems[p]),
                pltpu.async_copy(isc_hbm.at[pl.ds(_B + r0, _IDX_CHUNK)],
                                 jrows2[p], sems[p]),
            )

        pending = {0: fire(0)}
        for rnd in range(n_rounds):
            if rnd + 1 < n_rounds:
                pending[rnd + 1] = fire(rnd + 1)
            for c in pending.pop(rnd):
                c.wait()
            urows, irows, jrows = urows2[rnd % 2], irows2[rnd % 2], jrows2[rnd % 2]

            def body(blk, _):
                row = blk * _LANES + lane
                acc_i = [jnp.zeros((_LANES,), jnp.float32) for _ in range(4)]
                acc_j = [jnp.zeros((_LANES,), jnp.float32) for _ in range(4)]
                for d in range(_DIM):
                    col = jnp.full((_LANES,), d, jnp.int32)
                    ug = plsc.load_gather(urows, [row, col])
                    ig = plsc.load_gather(irows, [row, col])
                    jg = plsc.load_gather(jrows, [row, col])
                    acc_i[d % 4] = acc_i[d % 4] + ug * ig
                    acc_j[d % 4] = acc_j[d % 4] + ug * jg
                o = pl.ds(rnd * _IDX_CHUNK + blk * _LANES, _LANES)
                outi[o] = (acc_i[0] + acc_i[1]) + (acc_i[2] + acc_i[3])
                outj[o] = (acc_j[0] + acc_j[1]) + (acc_j[2] + acc_j[3])
                return _
            lax.fori_loop(0, _IDX_CHUNK // _LANES, body, None)

        pltpu.sync_copy(outi, out_i_hbm.at[pl.ds(base, b_per_w)])
        pltpu.sync_copy(outj, out_j_hbm.at[pl.ds(base, b_per_w)])

    return kern


@jax.jit
def kernel(user, item_i, item_j, user_emb_weight, item_emb_weight):
    extract = _extract_kernel()
    gather_dot = _gather_dot_kernel()

    ut = user_emb_weight.T
    it = item_emb_weight.T
    utail = ut[:, _VMAIN:].reshape(-1)
    itail = it[:, _VMAIN:].reshape(-1)

    usc, isc = extract(user, item_i, item_j, ut, it, utail, itail)
    rating_i, rating_j = gather_dot(usc, isc)
    return (rating_i, rating_j)
